# Initial kernel scaffold; baseline (speedup 1.0000x reference)
import jax
import jax.numpy as jnp
from jax.experimental import pallas as pl


def kernel(x, router_weight, router_bias, w13, w13_bias, w2, w2_bias):
    raise NotImplementedError("write your pallas kernel here")


# speedup vs baseline: 1.2444x; 1.2444x over previous
import functools

import jax
import jax.numpy as jnp
from jax import lax
from jax.experimental import pallas as pl
from jax.experimental.pallas import tpu as pltpu
from jax.experimental.pallas import tpu_sc as plsc

T = 2048
D = 768
F = 3072
E = 8
K = 2
BLK = 128
NB = (T * K) // BLK + E
NPAD = NB * BLK

_NC = 2
_NS = 16
_NW = _NC * _NS
_GROWS = NPAD // _NW
_CROWS = T // _NW

_SC_MESH = dict(core_axis_name="c", subcore_axis_name="s",
                num_cores=_NC, num_subcores=_NS)


def _router_body(x_ref, rw_ref, rb_ref, ids_ref, wts_ref):
    logits = lax.dot_general(x_ref[...], rw_ref[...],
                             (((1,), (1,)), ((), ())),
                             preferred_element_type=jnp.float32)
    logits = logits + rb_ref[...]
    lane = lax.broadcasted_iota(jnp.int32, (T, E), 1)
    neg = jnp.finfo(jnp.float32).min
    m1 = jnp.max(logits, axis=-1, keepdims=True)
    i1 = jnp.min(jnp.where(logits == m1, lane, E), axis=-1, keepdims=True)
    rest = jnp.where(lane == i1, neg, logits)
    m2 = jnp.max(rest, axis=-1, keepdims=True)
    i2 = jnp.min(jnp.where(rest == m2, lane, E), axis=-1, keepdims=True)
    z = jnp.exp(m2 - m1)
    w1 = 1.0 / (1.0 + z)
    ids_ref[...] = jnp.concatenate([i1, i2], axis=-1)
    wts_ref[...] = jnp.concatenate([w1, 1.0 - w1], axis=-1)


def _router(x, router_weight, router_bias):
    return pl.pallas_call(
        _router_body,
        out_shape=(jax.ShapeDtypeStruct((T, K), jnp.int32),
                   jax.ShapeDtypeStruct((T, K), jnp.float32)),
    )(x, router_weight, router_bias)


def _dispatch_meta(ids, wts):
    flat_e = ids.reshape(-1)
    flat_t = jnp.arange(T * K, dtype=jnp.int32) // K
    flat_w = wts.reshape(-1)
    onehot = (flat_e[:, None] == jnp.arange(E, dtype=flat_e.dtype)[None, :]
              ).astype(jnp.int32)
    cum = jnp.cumsum(onehot, axis=0)
    rank = jnp.take_along_axis(cum, flat_e[:, None].astype(jnp.int32),
                               axis=1)[:, 0] - 1
    counts = cum[-1]
    nblk = (counts + BLK - 1) // BLK
    blk_start = jnp.cumsum(nblk) - nblk
    base = (BLK * blk_start).astype(jnp.int32)
    dest = base[flat_e] + rank
    row_token = jnp.zeros((NPAD,), jnp.int32).at[dest].set(flat_t)
    row_weight = jnp.zeros((NPAD,), jnp.float32).at[dest].set(flat_w)
    block_expert = jnp.minimum(
        jnp.sum(jnp.arange(NB)[:, None] >= (blk_start + nblk)[None, :],
                axis=1), E - 1).astype(jnp.int32)
    slots = dest.reshape(T, K).astype(jnp.int32)
    return row_token, row_weight, block_expert, slots


@functools.cache
def _make_sc_gather():
    @functools.partial(
        pl.kernel,
        out_type=jax.ShapeDtypeStruct((NPAD, D), jnp.float32),
        mesh=plsc.VectorSubcoreMesh(**_SC_MESH),
        scratch_types=[pltpu.VMEM((_GROWS,), jnp.int32),
                       pltpu.VMEM((_GROWS, D), jnp.float32),
                       pltpu.SemaphoreType.DMA],
    )
    def _sc_gather(x_hbm, idx_hbm, out_hbm, idx_v, rows_v, sem):
        wid = lax.axis_index("s") * _NC + lax.axis_index("c")
        base = wid * _GROWS
        pltpu.sync_copy(idx_hbm.at[pl.ds(base, _GROWS)], idx_v)
        pltpu.async_copy(x_hbm.at[idx_v], rows_v, sem).wait()
        pltpu.sync_copy(rows_v, out_hbm.at[pl.ds(base, _GROWS)])

    return _sc_gather


def _gemm_body(be_ref, xs_ref, w13_ref, b13_ref, w2_ref, b2_ref, rw_ref,
               out_ref):
    xb = xs_ref[...]
    h = lax.dot_general(xb, w13_ref[0], (((1,), (1,)), ((), ())),
                        preferred_element_type=jnp.float32)
    h = h + b13_ref[0]
    gate = h[:, :F]
    up = h[:, F:]
    act = gate * jax.nn.sigmoid(gate) * up
    y = lax.dot_general(act, w2_ref[0], (((1,), (1,)), ((), ())),
                        preferred_element_type=jnp.float32)
    y = y + b2_ref[0]
    out_ref[...] = y * rw_ref[0]


def _grouped_gemm(block_expert, x_sorted, w13, w13_bias, w2, w2_bias,
                  row_weight):
    grid_spec = pltpu.PrefetchScalarGridSpec(
        num_scalar_prefetch=1,
        grid=(NB,),
        in_specs=[
            pl.BlockSpec((BLK, D), lambda b, be: (b, 0)),
            pl.BlockSpec((1, 2 * F, D), lambda b, be: (be[b], 0, 0)),
            pl.BlockSpec((1, 1, 2 * F), lambda b, be: (be[b], 0, 0)),
            pl.BlockSpec((1, D, F), lambda b, be: (be[b], 0, 0)),
            pl.BlockSpec((1, 1, D), lambda b, be: (be[b], 0, 0)),
            pl.BlockSpec((1, BLK, 1), lambda b, be: (b, 0, 0)),
        ],
        out_specs=pl.BlockSpec((BLK, D), lambda b, be: (b, 0)),
    )
    return pl.pallas_call(
        _gemm_body,
        grid_spec=grid_spec,
        out_shape=jax.ShapeDtypeStruct((NPAD, D), jnp.float32),
    )(block_expert, x_sorted,
      w13, w13_bias.reshape(E, 1, 2 * F),
      w2, w2_bias.reshape(E, 1, D),
      row_weight.reshape(NB, BLK, 1))


@functools.cache
def _make_sc_combine():
    @functools.partial(
        pl.kernel,
        out_type=jax.ShapeDtypeStruct((T, D), jnp.float32),
        mesh=plsc.VectorSubcoreMesh(**_SC_MESH),
        scratch_types=[pltpu.VMEM((_CROWS,), jnp.int32),
                       pltpu.VMEM((_CROWS,), jnp.int32),
                       pltpu.VMEM((_CROWS, D), jnp.float32),
                       pltpu.VMEM((_CROWS, D), jnp.float32),
                       pltpu.SemaphoreType.DMA,
                       pltpu.SemaphoreType.DMA],
    )
    def _sc_combine(ys_hbm, sa_hbm, sb_hbm, out_hbm, ia_v, ib_v, ba_v, bb_v,
                    sem_a, sem_b):
        wid = lax.axis_index("s") * _NC + lax.axis_index("c")
        base = wid * _CROWS
        pltpu.sync_copy(sa_hbm.at[pl.ds(base, _CROWS)], ia_v)
        pltpu.sync_copy(sb_hbm.at[pl.ds(base, _CROWS)], ib_v)
        ca = pltpu.async_copy(ys_hbm.at[ia_v], ba_v, sem_a)
        cb = pltpu.async_copy(ys_hbm.at[ib_v], bb_v, sem_b)
        ca.wait()
        cb.wait()
        nch = D // 16

        def body(i, carry):
            r = i // nch
            c = (i % nch) * 16
            ba_v[r, pl.ds(c, 16)] = ba_v[r, pl.ds(c, 16)] + bb_v[r, pl.ds(c, 16)]
            return carry

        lax.fori_loop(0, _CROWS * nch, body, 0)
        pltpu.sync_copy(ba_v, out_hbm.at[pl.ds(base, _CROWS)])

    return _sc_combine


def kernel(x, router_weight, router_bias, w13, w13_bias, w2, w2_bias):
    ids, wts = _router(x, router_weight, router_bias)
    row_token, row_weight, block_expert, slots = _dispatch_meta(ids, wts)
    x_sorted = _make_sc_gather()(x, row_token)
    y_sorted = _grouped_gemm(block_expert, x_sorted, w13, w13_bias,
                             w2, w2_bias, row_weight)
    out = _make_sc_combine()(y_sorted, slots[:, 0], slots[:, 1])
    return out


# speedup vs baseline: 1.2490x; 1.0037x over previous
import functools

import jax
import jax.numpy as jnp
from jax import lax
from jax.experimental import pallas as pl
from jax.experimental.pallas import tpu as pltpu
from jax.experimental.pallas import tpu_sc as plsc

T = 2048
D = 768
F = 3072
E = 8
K = 2
BLK = 128
NB = (T * K) // BLK + E
NPAD = NB * BLK

_NC = 2
_NS = 16
_NW = _NC * _NS
_GROWS = NPAD // _NW
_CROWS = T // _NW

_SC_MESH = dict(core_axis_name="c", subcore_axis_name="s",
                num_cores=_NC, num_subcores=_NS)


def _router_body(x_ref, rw_ref, rb_ref, ids_ref, wts_ref):
    logits = lax.dot_general(x_ref[...], rw_ref[...],
                             (((1,), (1,)), ((), ())),
                             preferred_element_type=jnp.float32)
    logits = logits + rb_ref[...]
    lane = lax.broadcasted_iota(jnp.int32, (T, E), 1)
    neg = jnp.finfo(jnp.float32).min
    m1 = jnp.max(logits, axis=-1, keepdims=True)
    i1 = jnp.min(jnp.where(logits == m1, lane, E), axis=-1, keepdims=True)
    rest = jnp.where(lane == i1, neg, logits)
    m2 = jnp.max(rest, axis=-1, keepdims=True)
    i2 = jnp.min(jnp.where(rest == m2, lane, E), axis=-1, keepdims=True)
    z = jnp.exp(m2 - m1)
    w1 = 1.0 / (1.0 + z)
    ids_ref[...] = jnp.concatenate([i1, i2], axis=-1)
    wts_ref[...] = jnp.concatenate([w1, 1.0 - w1], axis=-1)


def _router(x, router_weight, router_bias):
    return pl.pallas_call(
        _router_body,
        out_shape=(jax.ShapeDtypeStruct((T, K), jnp.int32),
                   jax.ShapeDtypeStruct((T, K), jnp.float32)),
    )(x, router_weight, router_bias)


def _dispatch_meta(ids, wts):
    flat_e = ids.reshape(-1)
    flat_t = jnp.arange(T * K, dtype=jnp.int32) // K
    flat_w = wts.reshape(-1)
    onehot = (flat_e[:, None] == jnp.arange(E, dtype=flat_e.dtype)[None, :]
              ).astype(jnp.int32)
    cum = jnp.cumsum(onehot, axis=0)
    rank = jnp.take_along_axis(cum, flat_e[:, None].astype(jnp.int32),
                               axis=1)[:, 0] - 1
    counts = cum[-1]
    nblk = (counts + BLK - 1) // BLK
    blk_start = jnp.cumsum(nblk) - nblk
    base = (BLK * blk_start).astype(jnp.int32)
    dest = base[flat_e] + rank
    row_token = jnp.zeros((NPAD,), jnp.int32).at[dest].set(flat_t)
    row_weight = jnp.zeros((NPAD,), jnp.float32).at[dest].set(flat_w)
    block_expert = jnp.minimum(
        jnp.sum(jnp.arange(NB)[:, None] >= (blk_start + nblk)[None, :],
                axis=1), E - 1).astype(jnp.int32)
    slots = dest.reshape(T, K).astype(jnp.int32)
    return row_token, row_weight, block_expert, slots


_GCH = 4
_GCROWS = _GROWS // _GCH


@functools.cache
def _make_sc_gather():
    @functools.partial(
        pl.kernel,
        out_type=jax.ShapeDtypeStruct((NPAD, D), jnp.float32),
        mesh=plsc.VectorSubcoreMesh(**_SC_MESH),
        scratch_types=[pltpu.VMEM((_GROWS,), jnp.int32),
                       pltpu.VMEM((_GCH, _GCROWS, D), jnp.float32),
                       [pltpu.SemaphoreType.DMA] * _GCH,
                       [pltpu.SemaphoreType.DMA] * _GCH],
    )
    def _sc_gather(x_hbm, idx_hbm, out_hbm, idx_v, bufs_v, rsems, wsems):
        wid = lax.axis_index("s") * _NC + lax.axis_index("c")
        base = wid * _GROWS
        pltpu.sync_copy(idx_hbm.at[pl.ds(base, _GROWS)], idx_v)
        reads = [
            pltpu.async_copy(x_hbm.at[idx_v.at[pl.ds(i * _GCROWS, _GCROWS)]],
                             bufs_v.at[i], rsems[i])
            for i in range(_GCH)
        ]
        writes = []
        for i in range(_GCH):
            reads[i].wait()
            writes.append(
                pltpu.async_copy(bufs_v.at[i],
                                 out_hbm.at[pl.ds(base + i * _GCROWS,
                                                  _GCROWS)], wsems[i]))
        for w in writes:
            w.wait()

    return _sc_gather


def _gemm_body(be_ref, xs_ref, w13_ref, b13_ref, w2_ref, b2_ref, rw_ref,
               out_ref):
    xb = xs_ref[...]
    h = lax.dot_general(xb, w13_ref[0], (((1,), (1,)), ((), ())),
                        preferred_element_type=jnp.float32)
    h = h + b13_ref[0]
    gate = h[:, :F]
    up = h[:, F:]
    act = gate * jax.nn.sigmoid(gate) * up
    y = lax.dot_general(act, w2_ref[0], (((1,), (1,)), ((), ())),
                        preferred_element_type=jnp.float32)
    y = y + b2_ref[0]
    out_ref[...] = y * rw_ref[0]


def _grouped_gemm(block_expert, x_sorted, w13, w13_bias, w2, w2_bias,
                  row_weight):
    grid_spec = pltpu.PrefetchScalarGridSpec(
        num_scalar_prefetch=1,
        grid=(NB,),
        in_specs=[
            pl.BlockSpec((BLK, D), lambda b, be: (b, 0)),
            pl.BlockSpec((1, 2 * F, D), lambda b, be: (be[b], 0, 0)),
            pl.BlockSpec((1, 1, 2 * F), lambda b, be: (be[b], 0, 0)),
            pl.BlockSpec((1, D, F), lambda b, be: (be[b], 0, 0)),
            pl.BlockSpec((1, 1, D), lambda b, be: (be[b], 0, 0)),
            pl.BlockSpec((1, BLK, 1), lambda b, be: (b, 0, 0)),
        ],
        out_specs=pl.BlockSpec((BLK, D), lambda b, be: (b, 0)),
    )
    return pl.pallas_call(
        _gemm_body,
        grid_spec=grid_spec,
        out_shape=jax.ShapeDtypeStruct((NPAD, D), jnp.float32),
    )(block_expert, x_sorted,
      w13, w13_bias.reshape(E, 1, 2 * F),
      w2, w2_bias.reshape(E, 1, D),
      row_weight.reshape(NB, BLK, 1))


@functools.cache
def _make_sc_combine():
    @functools.partial(
        pl.kernel,
        out_type=jax.ShapeDtypeStruct((T, D), jnp.float32),
        mesh=plsc.VectorSubcoreMesh(**_SC_MESH),
        scratch_types=[pltpu.VMEM((_CROWS,), jnp.int32),
                       pltpu.VMEM((_CROWS,), jnp.int32),
                       pltpu.VMEM((_CROWS, D), jnp.float32),
                       pltpu.VMEM((_CROWS, D), jnp.float32),
                       pltpu.SemaphoreType.DMA,
                       pltpu.SemaphoreType.DMA],
    )
    def _sc_combine(ys_hbm, sa_hbm, sb_hbm, out_hbm, ia_v, ib_v, ba_v, bb_v,
                    sem_a, sem_b):
        wid = lax.axis_index("s") * _NC + lax.axis_index("c")
        base = wid * _CROWS
        pltpu.sync_copy(sa_hbm.at[pl.ds(base, _CROWS)], ia_v)
        pltpu.sync_copy(sb_hbm.at[pl.ds(base, _CROWS)], ib_v)
        ca = pltpu.async_copy(ys_hbm.at[ia_v], ba_v, sem_a)
        cb = pltpu.async_copy(ys_hbm.at[ib_v], bb_v, sem_b)
        ca.wait()
        cb.wait()
        nch = D // 16

        def body(i, carry):
            r = i // nch
            c = (i % nch) * 16
            ba_v[r, pl.ds(c, 16)] = ba_v[r, pl.ds(c, 16)] + bb_v[r, pl.ds(c, 16)]
            return carry

        lax.fori_loop(0, _CROWS * nch, body, 0)
        pltpu.sync_copy(ba_v, out_hbm.at[pl.ds(base, _CROWS)])

    return _sc_combine


def kernel(x, router_weight, router_bias, w13, w13_bias, w2, w2_bias):
    ids, wts = _router(x, router_weight, router_bias)
    row_token, row_weight, block_expert, slots = _dispatch_meta(ids, wts)
    x_sorted = _make_sc_gather()(x, row_token)
    y_sorted = _grouped_gemm(block_expert, x_sorted, w13, w13_bias,
                             w2, w2_bias, row_weight)
    out = _make_sc_combine()(y_sorted, slots[:, 0], slots[:, 1])
    return out


# speedup vs baseline: 1.5565x; 1.2462x over previous
import functools

import jax
import jax.numpy as jnp
from jax import lax
from jax.experimental import pallas as pl
from jax.experimental.pallas import tpu as pltpu
from jax.experimental.pallas import tpu_sc as plsc

T = 2048
D = 768
F = 3072
E = 8
K = 2
A = T * K
BLK = 128
NB = A // BLK + E
NPAD = NB * BLK

_NC = 2
_NS = 16
_NW = _NC * _NS
_TPW = T // _NW

_SC_MESH = dict(core_axis_name="c", subcore_axis_name="s",
                num_cores=_NC, num_subcores=_NS)


def _router_body(x_ref, rw_ref, rb_ref, dest_ref, wts_ref, bexp_ref):
    logits = lax.dot_general(x_ref[...], rw_ref[...],
                             (((1,), (1,)), ((), ())),
                             preferred_element_type=jnp.float32)
    logits = logits + rb_ref[...]
    lane = lax.broadcasted_iota(jnp.int32, (T, E), 1)
    neg = jnp.finfo(jnp.float32).min
    m1 = jnp.max(logits, axis=-1, keepdims=True)
    i1 = jnp.min(jnp.where(logits == m1, lane, E), axis=-1, keepdims=True)
    rest = jnp.where(lane == i1, neg, logits)
    m2 = jnp.max(rest, axis=-1, keepdims=True)
    i2 = jnp.min(jnp.where(rest == m2, lane, E), axis=-1, keepdims=True)
    z = jnp.exp(m2 - m1)
    w1 = 1.0 / (1.0 + z)
    wts_ref[...] = jnp.broadcast_to(
        jnp.concatenate([w1, 1.0 - w1], axis=0), (A, 16))

    fe = jnp.concatenate([i1, i2], axis=0)
    alane = lax.broadcasted_iota(jnp.int32, (A, E), 1)
    oh = (fe == alane).astype(jnp.int32)
    cum = oh
    k = 1
    while k < A:
        cum = cum + jnp.concatenate(
            [jnp.zeros((k, E), jnp.int32), cum[:A - k, :]], axis=0)
        k *= 2
    rank = jnp.sum((cum - 1) * oh, axis=1, keepdims=True)
    counts = cum[A - 1:, :]
    nblk = (counts + BLK - 1) // BLK
    nblk_ba = jnp.broadcast_to(nblk, (A, E))
    base = BLK * jnp.sum(jnp.where(alane < fe, nblk_ba, 0),
                         axis=1, keepdims=True)
    dest_ref[...] = rank + base

    srow = lax.broadcasted_iota(jnp.int32, (E, E), 0)
    scol = lax.broadcasted_iota(jnp.int32, (E, E), 1)
    nblk_col = jnp.sum(jnp.broadcast_to(nblk, (E, E))
                       * (srow == scol).astype(jnp.int32),
                       axis=1, keepdims=True)
    bsn = jnp.sum(jnp.where(srow <= scol, jnp.broadcast_to(nblk_col, (E, E)),
                            0), axis=0, keepdims=True)
    bsn_b = jnp.broadcast_to(bsn, (NB, E))
    brow = lax.broadcasted_iota(jnp.int32, (NB, E), 0)
    bexp_ref[...] = jnp.minimum(
        jnp.sum((brow >= bsn_b).astype(jnp.int32), axis=1, keepdims=True),
        E - 1)


def _router(x, router_weight, router_bias):
    return pl.pallas_call(
        _router_body,
        out_shape=(jax.ShapeDtypeStruct((A, 1), jnp.int32),
                   jax.ShapeDtypeStruct((A, 16), jnp.float32),
                   jax.ShapeDtypeStruct((NB, 1), jnp.int32)),
    )(x, router_weight, router_bias)


@functools.cache
def _make_sc_dispatch():
    @functools.partial(
        pl.kernel,
        out_type=jax.ShapeDtypeStruct((NPAD, D), jnp.float32),
        mesh=plsc.VectorSubcoreMesh(**_SC_MESH),
        scratch_types=[pltpu.VMEM((_TPW,), jnp.int32),
                       pltpu.VMEM((_TPW,), jnp.int32),
                       pltpu.VMEM((_TPW, D), jnp.float32),
                       pltpu.SemaphoreType.DMA,
                       pltpu.SemaphoreType.DMA],
    )
    def _sc_dispatch(x_hbm, didx_hbm, out_hbm, ia_v, ib_v, rows_v,
                     sem_a, sem_b):
        wid = lax.axis_index("s") * _NC + lax.axis_index("c")
        pltpu.sync_copy(didx_hbm.at[wid], ia_v)
        pltpu.sync_copy(didx_hbm.at[_NW + wid], ib_v)
        pltpu.sync_copy(x_hbm.at[pl.ds(wid * _TPW, _TPW)], rows_v)
        ca = pltpu.async_copy(rows_v, out_hbm.at[ia_v], sem_a)
        cb = pltpu.async_copy(rows_v, out_hbm.at[ib_v], sem_b)
        ca.wait()
        cb.wait()

    return _sc_dispatch


def _gemm_body(be_ref, xs_ref, w13_ref, b13_ref, w2_ref, b2_ref, out_ref):
    xb = xs_ref[...]
    h = lax.dot_general(xb, w13_ref[0], (((1,), (1,)), ((), ())),
                        preferred_element_type=jnp.float32)
    h = h + b13_ref[0]
    gate = h[:, :F]
    up = h[:, F:]
    act = gate * jax.nn.sigmoid(gate) * up
    y = lax.dot_general(act, w2_ref[0], (((1,), (1,)), ((), ())),
                        preferred_element_type=jnp.float32)
    out_ref[...] = y + b2_ref[0]


def _grouped_gemm(block_expert, x_sorted, w13, w13_bias, w2, w2_bias):
    grid_spec = pltpu.PrefetchScalarGridSpec(
        num_scalar_prefetch=1,
        grid=(NB,),
        in_specs=[
            pl.BlockSpec((BLK, D), lambda b, be: (b, 0)),
            pl.BlockSpec((1, 2 * F, D), lambda b, be: (be[b], 0, 0)),
            pl.BlockSpec((1, 1, 2 * F), lambda b, be: (be[b], 0, 0)),
            pl.BlockSpec((1, D, F), lambda b, be: (be[b], 0, 0)),
            pl.BlockSpec((1, 1, D), lambda b, be: (be[b], 0, 0)),
        ],
        out_specs=pl.BlockSpec((BLK, D), lambda b, be: (b, 0)),
    )
    return pl.pallas_call(
        _gemm_body,
        grid_spec=grid_spec,
        out_shape=jax.ShapeDtypeStruct((NPAD, D), jnp.float32),
    )(block_expert, x_sorted,
      w13, w13_bias.reshape(E, 1, 2 * F),
      w2, w2_bias.reshape(E, 1, D))


@functools.cache
def _make_sc_combine():
    @functools.partial(
        pl.kernel,
        out_type=jax.ShapeDtypeStruct((T, D), jnp.float32),
        mesh=plsc.VectorSubcoreMesh(**_SC_MESH),
        scratch_types=[pltpu.VMEM((_TPW,), jnp.int32),
                       pltpu.VMEM((_TPW,), jnp.int32),
                       pltpu.VMEM((_TPW, 16), jnp.float32),
                       pltpu.VMEM((_TPW, 16), jnp.float32),
                       pltpu.VMEM((_TPW, D), jnp.float32),
                       pltpu.VMEM((_TPW, D), jnp.float32),
                       pltpu.SemaphoreType.DMA,
                       pltpu.SemaphoreType.DMA],
    )
    def _sc_combine(ys_hbm, didx_hbm, w_hbm, out_hbm, ia_v, ib_v, wa_v, wb_v,
                    ba_v, bb_v, sem_a, sem_b):
        wid = lax.axis_index("s") * _NC + lax.axis_index("c")
        pltpu.sync_copy(didx_hbm.at[wid], ia_v)
        pltpu.sync_copy(didx_hbm.at[_NW + wid], ib_v)
        pltpu.sync_copy(w_hbm.at[wid], wa_v)
        pltpu.sync_copy(w_hbm.at[_NW + wid], wb_v)
        ca = pltpu.async_copy(ys_hbm.at[ia_v], ba_v, sem_a)
        cb = pltpu.async_copy(ys_hbm.at[ib_v], bb_v, sem_b)
        ca.wait()
        cb.wait()
        nch = D // 16

        def body(i, carry):
            r = i // nch
            c = (i % nch) * 16
            wa = wa_v[r, :]
            wb = wb_v[r, :]
            ba_v[r, pl.ds(c, 16)] = (wa * ba_v[r, pl.ds(c, 16)]
                                     + wb * bb_v[r, pl.ds(c, 16)])
            return carry

        lax.fori_loop(0, _TPW * nch, body, 0)
        pltpu.sync_copy(ba_v, out_hbm.at[pl.ds(wid * _TPW, _TPW)])

    return _sc_combine


def kernel(x, router_weight, router_bias, w13, w13_bias, w2, w2_bias):
    dest, wts, block_expert = _router(x, router_weight, router_bias)
    didx = dest.reshape(2 * _NW, _TPW)
    wflat = wts.reshape(2 * _NW, _TPW, 16)
    x_sorted = _make_sc_dispatch()(x, didx)
    y_sorted = _grouped_gemm(block_expert.reshape(NB), x_sorted,
                             w13, w13_bias, w2, w2_bias)
    out = _make_sc_combine()(y_sorted, didx, wflat)
    return out
